# baseline (speedup 1.0000x reference)
import functools

import jax
import jax.numpy as jnp
from jax import lax
from jax.experimental import pallas as pl
from jax.experimental.pallas import tpu as pltpu
from jax.experimental.pallas import tpu_sc as plsc

N, L, H, D = 16384, 20, 128, 128
NC, NS = 2, 16
NW = NC * NS
RPW = N // NW
CH = 128
NCH = RPW // CH

_mesh = plsc.VectorSubcoreMesh(
    core_axis_name="c", subcore_axis_name="s",
    num_cores=NC, num_subcores=NS)


def _wid():
    return lax.axis_index("s") * NC + lax.axis_index("c")


@functools.partial(
    pl.kernel,
    out_type=(),
    mesh=_mesh,
    scratch_types=[
        pltpu.VMEM((NCH, CH), jnp.int32),
        pltpu.VMEM((RPW, H), jnp.float32),
        pltpu.SemaphoreType.DMA,
    ],
)
def _sc_scatter(mem_ref, idx_hbm, nexth_hbm, idx_v, rows_v, sem):
    base = _wid() * RPW
    pltpu.sync_copy(idx_hbm.at[_wid()], idx_v)
    pltpu.sync_copy(nexth_hbm.at[pl.ds(base, RPW)], rows_v)
    copies = [
        pltpu.async_copy(rows_v.at[pl.ds(c * CH, CH)],
                         mem_ref.at[idx_v.at[c]], sem)
        for c in range(NCH)
    ]
    for cp in copies:
        cp.wait()


@jax.jit
def kernel(memory, veh_idx, veh_repr, cust_repr, edge_emb, W_in, b_in,
           W_h, b_h):
    n, l, h = memory.shape
    flat_idx = (jnp.arange(n, dtype=jnp.int32) * l
                + veh_idx[:, 0].astype(jnp.int32))
    idx3 = flat_idx.reshape(NW, NCH, CH)
    out_ref = jax.new_ref(memory.reshape(n * l, h))
    _sc_scatter(out_ref, idx3, veh_repr[:, 0, :])
    return jax.freeze(out_ref).reshape(n, l, h)


# speedup vs baseline: 4.0804x; 4.0804x over previous
import functools

import jax
import jax.numpy as jnp
from jax import lax
from jax.experimental import pallas as pl
from jax.experimental.pallas import tpu as pltpu
from jax.experimental.pallas import tpu_sc as plsc

N, L, H, D = 16384, 20, 128, 128
NC, NS = 2, 16
NW = NC * NS
RPW = N // NW
CH = 128
NCH = RPW // CH

_mesh = plsc.VectorSubcoreMesh(
    core_axis_name="c", subcore_axis_name="s",
    num_cores=NC, num_subcores=NS)


def _wid():
    return lax.axis_index("s") * NC + lax.axis_index("c")


@functools.partial(
    pl.kernel,
    out_type=jax.ShapeDtypeStruct((N, H), jnp.float32),
    mesh=_mesh,
    scratch_types=[
        pltpu.VMEM((NCH, CH), jnp.int32),
        pltpu.VMEM((RPW, H), jnp.float32),
        pltpu.SemaphoreType.DMA,
    ],
)
def _sc_gather(table_hbm, idx_hbm, out_hbm, idx_v, rows_v, sem):
    base = _wid() * RPW
    pltpu.sync_copy(idx_hbm.at[_wid()], idx_v)
    copies = [
        pltpu.async_copy(table_hbm.at[idx_v.at[c]],
                         rows_v.at[pl.ds(c * CH, CH)], sem)
        for c in range(NCH)
    ]
    for cp in copies:
        cp.wait()
    pltpu.sync_copy(rows_v, out_hbm.at[pl.ds(base, RPW)])


@functools.partial(
    pl.kernel,
    out_type=(),
    mesh=_mesh,
    scratch_types=[
        pltpu.VMEM((NCH, CH), jnp.int32),
        pltpu.VMEM((RPW, H), jnp.float32),
        pltpu.SemaphoreType.DMA,
    ],
)
def _sc_scatter(table_ref, idx_hbm, nexth_hbm, idx_v, rows_v, sem):
    base = _wid() * RPW
    pltpu.sync_copy(idx_hbm.at[_wid()], idx_v)
    pltpu.sync_copy(nexth_hbm.at[pl.ds(base, RPW)], rows_v)
    copies = [
        pltpu.async_copy(rows_v.at[pl.ds(c * CH, CH)],
                         table_ref.at[idx_v.at[c]], sem)
        for c in range(NCH)
    ]
    for cp in copies:
        cp.wait()


BM = 2048


def _mlp_body(veh_ref, cust_ref, edge_ref, curh_ref, win_ref, bias_ref,
              wh_ref, out_ref):
    pre = jnp.dot(veh_ref[...], win_ref[0:D, :],
                  preferred_element_type=jnp.float32)
    pre += jnp.dot(cust_ref[...], win_ref[D:2 * D, :],
                   preferred_element_type=jnp.float32)
    pre += jnp.dot(edge_ref[...], win_ref[2 * D:3 * D, :],
                   preferred_element_type=jnp.float32)
    pre += jnp.dot(curh_ref[...], wh_ref[...],
                   preferred_element_type=jnp.float32)
    out_ref[...] = jnp.tanh(pre + bias_ref[...])


def _tc_mlp(veh, cust, edge, cur_h, W_in, bias, W_h):
    row = lambda i: (i, 0)
    full = lambda i: (0, 0)
    return pl.pallas_call(
        _mlp_body,
        grid=(N // BM,),
        in_specs=[
            pl.BlockSpec((BM, D), row),
            pl.BlockSpec((BM, D), row),
            pl.BlockSpec((BM, D), row),
            pl.BlockSpec((BM, H), row),
            pl.BlockSpec((3 * D, H), full),
            pl.BlockSpec((1, H), full),
            pl.BlockSpec((D, H), full),
        ],
        out_specs=pl.BlockSpec((BM, H), row),
        out_shape=jax.ShapeDtypeStruct((N, H), jnp.float32),
    )(veh, cust, edge, cur_h, W_in, bias, W_h)


@jax.jit
def kernel(memory, veh_idx, veh_repr, cust_repr, edge_emb, W_in, b_in,
           W_h, b_h):
    n, l, h = memory.shape
    tbl = memory.transpose(1, 0, 2).reshape(l * n, h)
    flat_idx = (veh_idx[:, 0].astype(jnp.int32) * n
                + jnp.arange(n, dtype=jnp.int32))
    idx3 = flat_idx.reshape(NW, NCH, CH)
    cur_h = _sc_gather(tbl, idx3)
    next_h = _tc_mlp(veh_repr[:, 0, :], cust_repr[:, 0, :],
                     edge_emb[:, 0, 0, :], cur_h,
                     W_in, (b_in + b_h).reshape(1, h), W_h)
    out_ref = jax.new_ref(tbl)
    _sc_scatter(out_ref, idx3, next_h)
    return jax.freeze(out_ref).reshape(l, n, h).transpose(1, 0, 2)


# speedup vs baseline: 4.7154x; 1.1556x over previous
import jax
import jax.numpy as jnp
from jax.experimental import pallas as pl

L, H, D = 20, 128, 128
BN = 512


def _body(vi_ref, veh_ref, cust_ref, edge_ref, win_ref, bias_ref, wh_ref,
          mem_ref, out_ref):
    mem = mem_ref[...]
    vi = vi_ref[...]
    cur_h = jnp.zeros((BN, H), jnp.float32)
    for s in range(L):
        cur_h += jnp.where(vi == s, mem[s], 0.0)
    pre = jnp.dot(veh_ref[...], win_ref[0:D, :],
                  preferred_element_type=jnp.float32)
    pre += jnp.dot(cust_ref[...], win_ref[D:2 * D, :],
                   preferred_element_type=jnp.float32)
    pre += jnp.dot(edge_ref[...], win_ref[2 * D:3 * D, :],
                   preferred_element_type=jnp.float32)
    pre += jnp.dot(cur_h, wh_ref[...], preferred_element_type=jnp.float32)
    next_h = jnp.tanh(pre + bias_ref[...])
    for s in range(L):
        out_ref[s] = jnp.where(vi == s, next_h, mem[s])


@jax.jit
def kernel(memory, veh_idx, veh_repr, cust_repr, edge_emb, W_in, b_in,
           W_h, b_h):
    n, l, h = memory.shape
    grid = n // BN
    bias = (b_in + b_h).reshape(1, h)
    row = lambda i: (i, 0)
    slab = lambda i: (0, i, 0)
    full = lambda i: (0, 0)
    out = pl.pallas_call(
        _body,
        grid=(grid,),
        in_specs=[
            pl.BlockSpec((BN, 1), row),
            pl.BlockSpec((BN, D), row),
            pl.BlockSpec((BN, D), row),
            pl.BlockSpec((BN, D), row),
            pl.BlockSpec((3 * D, h), full),
            pl.BlockSpec((1, h), full),
            pl.BlockSpec((D, h), full),
            pl.BlockSpec((l, BN, h), slab),
        ],
        out_specs=pl.BlockSpec((l, BN, h), slab),
        out_shape=jax.ShapeDtypeStruct((l, n, h), memory.dtype),
    )(veh_idx, veh_repr[:, 0, :], cust_repr[:, 0, :], edge_emb[:, 0, 0, :],
      W_in, bias, W_h, memory.transpose(1, 0, 2))
    return out.transpose(1, 0, 2)
